# Initial kernel scaffold; baseline (speedup 1.0000x reference)
import functools

import jax
import jax.numpy as jnp
from jax import lax
from jax.experimental import pallas as pl
from jax.experimental.pallas import tpu as pltpu
from jax.experimental.pallas import tpu_sc as plsc

N_NODES = 10000
N_EDGES = 320000
D_FEAT = 128
D_EDGE = 16
UNITS = 128

SLICE = 208
MSG = 192
NSUB = 16
EDGES_PER_SUB = N_EDGES // NSUB
BLK = 80
NBLK = EDGES_PER_SUB // BLK
ZROWS = 125
ROWS_PER_SUB = N_NODES // NSUB


def _node_tables(xa, kp0, kp1):
    def body(x_ref, k0_ref, k1_ref, o_ref):
        x = x_ref[...]
        o_ref[0] = jnp.dot(x, k0_ref[...], preferred_element_type=jnp.float32)
        o_ref[1] = jnp.dot(x, k1_ref[...], preferred_element_type=jnp.float32)

    bn = 1000
    return pl.pallas_call(
        body,
        grid=(N_NODES // bn,),
        in_specs=[
            pl.BlockSpec((bn, 136), lambda i: (i, 0)),
            pl.BlockSpec((136, SLICE), lambda i: (0, 0)),
            pl.BlockSpec((136, SLICE), lambda i: (0, 0)),
        ],
        out_specs=pl.BlockSpec((2, bn, SLICE), lambda i: (0, i, 0)),
        out_shape=jax.ShapeDtypeStruct((2, N_NODES, SLICE), jnp.float32),
    )(xa, kp0, kp1)


def _edge_tables(e_feats, we1, be1, we2a, be2a, we2b, be2b):
    def body(e_ref, w1_ref, b1_ref, w2a_ref, b2a_ref, w2b_ref, b2b_ref, o_ref):
        h = jnp.maximum(
            jnp.dot(e_ref[...], w1_ref[...], preferred_element_type=jnp.float32)
            + b1_ref[...], 0.0)
        o_ref[0] = jnp.dot(h, w2a_ref[...], preferred_element_type=jnp.float32) + b2a_ref[...]
        o_ref[1] = jnp.dot(h, w2b_ref[...], preferred_element_type=jnp.float32) + b2b_ref[...]

    be = 2000
    return pl.pallas_call(
        body,
        grid=(N_EDGES // be,),
        in_specs=[
            pl.BlockSpec((be, D_EDGE), lambda i: (i, 0)),
            pl.BlockSpec((D_EDGE, UNITS), lambda i: (0, 0)),
            pl.BlockSpec((1, UNITS), lambda i: (0, 0)),
            pl.BlockSpec((UNITS, SLICE), lambda i: (0, 0)),
            pl.BlockSpec((1, SLICE), lambda i: (0, 0)),
            pl.BlockSpec((UNITS, SLICE), lambda i: (0, 0)),
            pl.BlockSpec((1, SLICE), lambda i: (0, 0)),
        ],
        out_specs=pl.BlockSpec((2, be, SLICE), lambda i: (0, i, 0)),
        out_shape=jax.ShapeDtypeStruct((2, N_EDGES, SLICE), jnp.float32),
    )(e_feats, we1, be1, we2a, be2a, we2b, be2b)


def _sc_aggregate(y_flat, w_flat, src, dst):
    mesh = plsc.VectorSubcoreMesh(core_axis_name="c", subcore_axis_name="s")

    @functools.partial(
        pl.kernel,
        mesh=mesh,
        out_type=jax.ShapeDtypeStruct((2 * N_NODES, SLICE), jnp.float32),
        scratch_types=[
            pltpu.VMEM((BLK,), jnp.int32),
            pltpu.VMEM((BLK,), jnp.int32),
            pltpu.VMEM((BLK, SLICE), jnp.float32),
            pltpu.VMEM((BLK, SLICE), jnp.float32),
            pltpu.VMEM((ZROWS, SLICE), jnp.float32),
            pltpu.VMEM_SHARED((N_NODES, SLICE), jnp.float32),
            pltpu.SemaphoreType.DMA,
        ],
    )
    def k(y_hbm, w_hbm, src_hbm, dst_hbm, acc_hbm,
          src_v, dst_v, y_v, w_v, z_v, acc_sh, sem):
        cid = lax.axis_index("c")
        sid = lax.axis_index("s")

        @pl.loop(0, ZROWS)
        def _(r):
            @pl.loop(0, SLICE, step=16)
            def _(c0):
                z_v.at[r, pl.ds(c0, 16)][...] = jnp.zeros((16,), jnp.float32)

        row0 = sid * ROWS_PER_SUB
        for kk in range(ROWS_PER_SUB // ZROWS):
            pltpu.sync_copy(z_v, acc_sh.at[pl.ds(row0 + kk * ZROWS, ZROWS), :])
        plsc.subcore_barrier()

        ebase = sid * EDGES_PER_SUB

        @pl.loop(0, NBLK)
        def _(g):
            base = ebase + g * BLK
            pltpu.sync_copy(src_hbm.at[pl.ds(base, BLK)], src_v)
            pltpu.sync_copy(dst_hbm.at[pl.ds(base, BLK)], dst_v)

            @pl.loop(0, BLK, step=16)
            def _(i0):
                src_v.at[pl.ds(i0, 16)][...] = (
                    src_v.at[pl.ds(i0, 16)][...] + cid * N_NODES)

            pltpu.async_copy(y_hbm.at[src_v], y_v, sem).wait()
            pltpu.sync_copy(w_hbm.at[pl.ds(cid * N_EDGES + base, BLK), :], w_v)

            @pl.loop(0, BLK)
            def _(r):
                @pl.loop(0, SLICE, step=16)
                def _(c0):
                    w_v.at[r, pl.ds(c0, 16)][...] = (
                        w_v.at[r, pl.ds(c0, 16)][...]
                        * y_v.at[r, pl.ds(c0, 16)][...])

            pltpu.sync_copy(w_v, acc_sh.at[dst_v], add=True)

        plsc.subcore_barrier()

        out0 = cid * N_NODES + row0
        for kk in range(ROWS_PER_SUB // ZROWS):
            pltpu.sync_copy(acc_sh.at[pl.ds(row0 + kk * ZROWS, ZROWS), :],
                            acc_hbm.at[pl.ds(out0 + kk * ZROWS, ZROWS), :])

    return k(y_flat, w_flat, src, dst)


def _fusion(acc0, acc1, wf0, wf1, bf2):
    def body(a0_ref, a1_ref, w0_ref, w1_ref, b_ref, o_ref):
        a0 = a0_ref[...]
        a1 = a1_ref[...]
        s = (jnp.dot(a0, w0_ref[...], preferred_element_type=jnp.float32)
             + jnp.dot(a1, w1_ref[...], preferred_element_type=jnp.float32))
        counts = a0[:, MSG:MSG + 1]
        denom = jnp.maximum(counts, 1.0)
        o_ref[...] = jnp.maximum(s / denom + b_ref[...], 0.0)

    bn = 1000
    return pl.pallas_call(
        body,
        grid=(N_NODES // bn,),
        in_specs=[
            pl.BlockSpec((bn, SLICE), lambda i: (i, 0)),
            pl.BlockSpec((bn, SLICE), lambda i: (i, 0)),
            pl.BlockSpec((SLICE, UNITS), lambda i: (0, 0)),
            pl.BlockSpec((SLICE, UNITS), lambda i: (0, 0)),
            pl.BlockSpec((1, UNITS), lambda i: (0, 0)),
        ],
        out_specs=pl.BlockSpec((bn, UNITS), lambda i: (i, 0)),
        out_shape=jax.ShapeDtypeStruct((N_NODES, UNITS), jnp.float32),
    )(acc0, acc1, wf0, wf1, bf2)


@jax.jit
def kernel(node_features, edge_indices, edge_features,
           K0, K1, K2, We1, be1, We2, be2, Wf, bf):
    f32 = jnp.float32
    src = edge_indices[0].astype(jnp.int32)
    dst = edge_indices[1].astype(jnp.int32)

    kcat = jnp.concatenate([K0, K1, K2], axis=1)
    xa = jnp.concatenate(
        [node_features,
         jnp.ones((N_NODES, 1), f32),
         jnp.zeros((N_NODES, 7), f32)], axis=1)

    def make_kp(cols):
        kp = jnp.zeros((136, SLICE), f32)
        kp = kp.at[:D_FEAT, :MSG].set(cols)
        kp = kp.at[D_FEAT, MSG].set(1.0)
        return kp

    kp0 = make_kp(kcat[:, :MSG])
    kp1 = make_kp(kcat[:, MSG:])

    def make_w2(cols, bcols):
        w2 = jnp.zeros((UNITS, SLICE), f32).at[:, :MSG].set(cols)
        b2 = jnp.zeros((1, SLICE), f32).at[0, :MSG].set(bcols)
        b2 = b2.at[0, MSG].set(1.0)
        return w2, b2

    we2a, be2a = make_w2(We2[:, :MSG], be2[:MSG])
    we2b, be2b = make_w2(We2[:, MSG:], be2[MSG:])

    wf0 = jnp.zeros((SLICE, UNITS), f32).at[:MSG, :].set(Wf[:MSG, :])
    wf1 = jnp.zeros((SLICE, UNITS), f32).at[:MSG, :].set(Wf[MSG:, :])
    bf2 = bf.reshape(1, UNITS)

    y = _node_tables(xa, kp0, kp1)
    w = _edge_tables(edge_features, We1, be1.reshape(1, UNITS),
                     we2a, be2a, we2b, be2b)

    acc = _sc_aggregate(y.reshape(2 * N_NODES, SLICE),
                        w.reshape(2 * N_EDGES, SLICE), src, dst)
    acc = acc.reshape(2, N_NODES, SLICE)

    return _fusion(acc[0], acc[1], wf0, wf1, bf2)


# speedup vs baseline: 3.8484x; 3.8484x over previous
import functools

import jax
import jax.numpy as jnp
from jax import lax
from jax.experimental import pallas as pl
from jax.experimental.pallas import tpu as pltpu
from jax.experimental.pallas import tpu_sc as plsc

N_NODES = 10000
N_EDGES = 320000
D_FEAT = 128
D_EDGE = 16
UNITS = 128
W3 = 384

NCORE = 2
NSUB = 16
NWORK = NCORE * NSUB
EDGES_PER_W = N_EDGES // NWORK
BLK = 80
NBLK = EDGES_PER_W // BLK
NPAD = 10240
STRIPE = NPAD // NSUB
CH = 128
NCH = STRIPE // CH


def _node_tables(x, kcat):
    def body(x_ref, k_ref, o_ref):
        o_ref[...] = jnp.dot(x_ref[...], k_ref[...],
                             preferred_element_type=jnp.float32)

    bn = 1000
    return pl.pallas_call(
        body,
        grid=(N_NODES // bn,),
        in_specs=[
            pl.BlockSpec((bn, D_FEAT), lambda i: (i, 0)),
            pl.BlockSpec((D_FEAT, W3), lambda i: (0, 0)),
        ],
        out_specs=pl.BlockSpec((bn, W3), lambda i: (i, 0)),
        out_shape=jax.ShapeDtypeStruct((N_NODES, W3), jnp.float32),
    )(x, kcat)


def _sc_gather(y, src):
    mesh = plsc.VectorSubcoreMesh(core_axis_name="c", subcore_axis_name="s")

    @functools.partial(
        pl.kernel,
        mesh=mesh,
        out_type=jax.ShapeDtypeStruct((N_EDGES, W3), jnp.float32),
        scratch_types=[
            pltpu.VMEM((BLK,), jnp.int32),
            pltpu.VMEM((BLK, W3), jnp.float32),
            pltpu.SemaphoreType.DMA,
        ],
    )
    def k(y_hbm, src_hbm, g_hbm, src_v, g_v, sem):
        cid = lax.axis_index("c")
        sid = lax.axis_index("s")
        ebase = (cid * NSUB + sid) * EDGES_PER_W

        @pl.loop(0, NBLK)
        def _(g):
            base = ebase + g * BLK
            pltpu.sync_copy(src_hbm.at[pl.ds(base, BLK)], src_v)
            pltpu.async_copy(y_hbm.at[src_v], g_v, sem).wait()
            pltpu.sync_copy(g_v, g_hbm.at[pl.ds(base, BLK), :])

    return k(y, src)


def _sc_counts(dst):
    mesh = plsc.VectorSubcoreMesh(core_axis_name="c", subcore_axis_name="s")

    @functools.partial(
        pl.kernel,
        mesh=mesh,
        out_type=jax.ShapeDtypeStruct((NCORE * NPAD, 128), jnp.float32),
        scratch_types=[
            pltpu.VMEM((BLK,), jnp.int32),
            pltpu.VMEM((BLK, 128), jnp.float32),
            pltpu.VMEM((CH, 128), jnp.float32),
            pltpu.VMEM_SHARED((NPAD, 128), jnp.float32),
        ],
    )
    def k(dst_hbm, cnt_hbm, dst_v, ones_v, z_v, cnt_sh):
        cid = lax.axis_index("c")
        sid = lax.axis_index("s")

        @pl.loop(0, CH)
        def _(r):
            @pl.loop(0, 128, step=16)
            def _(c0):
                z_v.at[r, pl.ds(c0, 16)][...] = jnp.zeros((16,), jnp.float32)

        @pl.loop(0, BLK)
        def _(r):
            @pl.loop(0, 128, step=16)
            def _(c0):
                ones_v.at[r, pl.ds(c0, 16)][...] = jnp.ones((16,), jnp.float32)

        row0 = sid * STRIPE
        for kk in range(NCH):
            pltpu.sync_copy(z_v, cnt_sh.at[pl.ds(row0 + kk * CH, CH), :])
        plsc.subcore_barrier()

        ebase = (cid * NSUB + sid) * EDGES_PER_W

        @pl.loop(0, NBLK)
        def _(g):
            base = ebase + g * BLK
            pltpu.sync_copy(dst_hbm.at[pl.ds(base, BLK)], dst_v)
            pltpu.sync_copy(ones_v, cnt_sh.at[dst_v], add=True)

        plsc.subcore_barrier()

        out0 = cid * NPAD + row0
        for kk in range(NCH):
            pltpu.sync_copy(cnt_sh.at[pl.ds(row0 + kk * CH, CH), :],
                            cnt_hbm.at[pl.ds(out0 + kk * CH, CH), :])

    return k(dst)


def _edge_stage(e_feats, g, we1, be1, we2, be2, wf):
    def body(e_ref, g_ref, w1_ref, b1_ref, w2_ref, b2_ref, wf_ref, o_ref):
        h = jnp.maximum(
            jnp.dot(e_ref[...], w1_ref[...], preferred_element_type=jnp.float32)
            + b1_ref[...], 0.0)
        w = jnp.dot(h, w2_ref[...], preferred_element_type=jnp.float32) + b2_ref[...]
        o_ref[...] = jnp.dot(w * g_ref[...], wf_ref[...],
                             preferred_element_type=jnp.float32)

    be = 2000
    return pl.pallas_call(
        body,
        grid=(N_EDGES // be,),
        in_specs=[
            pl.BlockSpec((be, D_EDGE), lambda i: (i, 0)),
            pl.BlockSpec((be, W3), lambda i: (i, 0)),
            pl.BlockSpec((D_EDGE, UNITS), lambda i: (0, 0)),
            pl.BlockSpec((1, UNITS), lambda i: (0, 0)),
            pl.BlockSpec((UNITS, W3), lambda i: (0, 0)),
            pl.BlockSpec((1, W3), lambda i: (0, 0)),
            pl.BlockSpec((W3, UNITS), lambda i: (0, 0)),
        ],
        out_specs=pl.BlockSpec((be, UNITS), lambda i: (i, 0)),
        out_shape=jax.ShapeDtypeStruct((N_EDGES, UNITS), jnp.float32),
    )(e_feats, g, we1, be1, we2, be2, wf)


def _sc_scatter(z, dst):
    mesh = plsc.VectorSubcoreMesh(core_axis_name="c", subcore_axis_name="s")

    @functools.partial(
        pl.kernel,
        mesh=mesh,
        out_type=jax.ShapeDtypeStruct((NCORE * NPAD, 128), jnp.float32),
        scratch_types=[
            pltpu.VMEM((BLK,), jnp.int32),
            pltpu.VMEM((BLK, 128), jnp.float32),
            pltpu.VMEM((CH, 128), jnp.float32),
            pltpu.VMEM_SHARED((NPAD, 128), jnp.float32),
        ],
    )
    def k(z_hbm, dst_hbm, acc_hbm, dst_v, z_v, zz_v, acc_sh):
        cid = lax.axis_index("c")
        sid = lax.axis_index("s")

        @pl.loop(0, CH)
        def _(r):
            @pl.loop(0, 128, step=16)
            def _(c0):
                zz_v.at[r, pl.ds(c0, 16)][...] = jnp.zeros((16,), jnp.float32)

        row0 = sid * STRIPE
        for kk in range(NCH):
            pltpu.sync_copy(zz_v, acc_sh.at[pl.ds(row0 + kk * CH, CH), :])
        plsc.subcore_barrier()

        ebase = (cid * NSUB + sid) * EDGES_PER_W

        @pl.loop(0, NBLK)
        def _(g):
            base = ebase + g * BLK
            pltpu.sync_copy(dst_hbm.at[pl.ds(base, BLK)], dst_v)
            pltpu.sync_copy(z_hbm.at[pl.ds(base, BLK), :], z_v)
            pltpu.sync_copy(z_v, acc_sh.at[dst_v], add=True)

        plsc.subcore_barrier()

        out0 = cid * NPAD + row0
        for kk in range(NCH):
            pltpu.sync_copy(acc_sh.at[pl.ds(row0 + kk * CH, CH), :],
                            acc_hbm.at[pl.ds(out0 + kk * CH, CH), :])

    return k(z, dst)


def _fusion(acc0, acc1, cnt0, cnt1, bf2):
    def body(a0_ref, a1_ref, c0_ref, c1_ref, b_ref, o_ref):
        s = a0_ref[...] + a1_ref[...]
        counts = c0_ref[...][:, 0:1] + c1_ref[...][:, 0:1]
        denom = jnp.maximum(counts, 1.0)
        o_ref[...] = jnp.maximum(s / denom + b_ref[...], 0.0)

    bn = 1024
    return pl.pallas_call(
        body,
        grid=(NPAD // bn,),
        in_specs=[
            pl.BlockSpec((bn, UNITS), lambda i: (i, 0)),
            pl.BlockSpec((bn, UNITS), lambda i: (i, 0)),
            pl.BlockSpec((bn, 128), lambda i: (i, 0)),
            pl.BlockSpec((bn, 128), lambda i: (i, 0)),
            pl.BlockSpec((1, UNITS), lambda i: (0, 0)),
        ],
        out_specs=pl.BlockSpec((bn, UNITS), lambda i: (i, 0)),
        out_shape=jax.ShapeDtypeStruct((NPAD, UNITS), jnp.float32),
    )(acc0, acc1, cnt0, cnt1, bf2)


@jax.jit
def kernel(node_features, edge_indices, edge_features,
           K0, K1, K2, We1, be1, We2, be2, Wf, bf):
    src = edge_indices[0].astype(jnp.int32)
    dst = edge_indices[1].astype(jnp.int32)

    kcat = jnp.concatenate([K0, K1, K2], axis=1)

    y = _node_tables(node_features, kcat)
    g = _sc_gather(y, src)
    cnt = _sc_counts(dst)
    z = _edge_stage(edge_features, g, We1, be1.reshape(1, UNITS),
                    We2, be2.reshape(1, W3), Wf)
    acc = _sc_scatter(z, dst)

    out = _fusion(acc[:NPAD], acc[NPAD:], cnt[:NPAD], cnt[NPAD:],
                  bf.reshape(1, UNITS))
    return out[:N_NODES]


# speedup vs baseline: 4.2296x; 1.0990x over previous
import functools

import jax
import jax.numpy as jnp
from jax import lax
from jax.experimental import pallas as pl
from jax.experimental.pallas import tpu as pltpu
from jax.experimental.pallas import tpu_sc as plsc

N_NODES = 10000
N_EDGES = 320000
D_FEAT = 128
D_EDGE = 16
UNITS = 128
W3 = 384

NCORE = 2
NSUB = 16
NWORK = NCORE * NSUB
EDGES_PER_W = N_EDGES // NWORK
BLK = 80
NBLK = EDGES_PER_W // BLK
NPAD = 10240
STRIPE = NPAD // NSUB
CH = 128
NCH = STRIPE // CH


def _node_tables(x, kcat):
    def body(x_ref, k_ref, o_ref):
        o_ref[...] = jnp.dot(x_ref[...], k_ref[...],
                             preferred_element_type=jnp.float32)

    bn = 1000
    return pl.pallas_call(
        body,
        grid=(N_NODES // bn,),
        in_specs=[
            pl.BlockSpec((bn, D_FEAT), lambda i: (i, 0)),
            pl.BlockSpec((D_FEAT, W3), lambda i: (0, 0)),
        ],
        out_specs=pl.BlockSpec((bn, W3), lambda i: (i, 0)),
        out_shape=jax.ShapeDtypeStruct((N_NODES, W3), jnp.float32),
    )(x, kcat)


def _sc_gather(y, src):
    mesh = plsc.VectorSubcoreMesh(core_axis_name="c", subcore_axis_name="s")

    @functools.partial(
        pl.kernel,
        mesh=mesh,
        out_type=jax.ShapeDtypeStruct((N_EDGES, W3), jnp.float32),
        scratch_types=[
            pltpu.VMEM((BLK,), jnp.int32),
            pltpu.VMEM((BLK,), jnp.int32),
            pltpu.VMEM((BLK, W3), jnp.float32),
            pltpu.VMEM((BLK, W3), jnp.float32),
            pltpu.SemaphoreType.DMA,
            pltpu.SemaphoreType.DMA,
        ],
    )
    def k(y_hbm, src_hbm, g_hbm, src_v0, src_v1, g_v0, g_v1, sem0, sem1):
        cid = lax.axis_index("c")
        sid = lax.axis_index("s")
        ebase = (cid * NSUB + sid) * EDGES_PER_W

        pltpu.sync_copy(src_hbm.at[pl.ds(ebase, BLK)], src_v0)
        pltpu.make_async_copy(y_hbm.at[src_v0], g_v0, sem0).start()

        @pl.loop(0, NBLK - 1, step=2)
        def _(g):
            base1 = ebase + (g + 1) * BLK
            pltpu.sync_copy(src_hbm.at[pl.ds(base1, BLK)], src_v1)
            pltpu.make_async_copy(y_hbm.at[src_v1], g_v1, sem1).start()
            pltpu.make_async_copy(y_hbm.at[src_v0], g_v0, sem0).wait()
            pltpu.sync_copy(g_v0, g_hbm.at[pl.ds(ebase + g * BLK, BLK), :])

            base2 = ebase + (g + 2) * BLK
            pltpu.sync_copy(src_hbm.at[pl.ds(base2, BLK)], src_v0)
            pltpu.make_async_copy(y_hbm.at[src_v0], g_v0, sem0).start()
            pltpu.make_async_copy(y_hbm.at[src_v1], g_v1, sem1).wait()
            pltpu.sync_copy(g_v1, g_hbm.at[pl.ds(base1, BLK), :])

        pltpu.make_async_copy(y_hbm.at[src_v0], g_v0, sem0).wait()
        pltpu.sync_copy(
            g_v0, g_hbm.at[pl.ds(ebase + (NBLK - 1) * BLK, BLK), :])

    return k(y, src)


def _sc_counts(dst):
    mesh = plsc.VectorSubcoreMesh(core_axis_name="c", subcore_axis_name="s")

    @functools.partial(
        pl.kernel,
        mesh=mesh,
        out_type=jax.ShapeDtypeStruct((NCORE * NPAD, 128), jnp.float32),
        scratch_types=[
            pltpu.VMEM((BLK,), jnp.int32),
            pltpu.VMEM((BLK, 128), jnp.float32),
            pltpu.VMEM((CH, 128), jnp.float32),
            pltpu.VMEM_SHARED((NPAD, 128), jnp.float32),
        ],
    )
    def k(dst_hbm, cnt_hbm, dst_v, ones_v, z_v, cnt_sh):
        cid = lax.axis_index("c")
        sid = lax.axis_index("s")

        @pl.loop(0, CH)
        def _(r):
            @pl.loop(0, 128, step=16)
            def _(c0):
                z_v.at[r, pl.ds(c0, 16)][...] = jnp.zeros((16,), jnp.float32)

        @pl.loop(0, BLK)
        def _(r):
            @pl.loop(0, 128, step=16)
            def _(c0):
                ones_v.at[r, pl.ds(c0, 16)][...] = jnp.ones((16,), jnp.float32)

        row0 = sid * STRIPE
        for kk in range(NCH):
            pltpu.sync_copy(z_v, cnt_sh.at[pl.ds(row0 + kk * CH, CH), :])
        plsc.subcore_barrier()

        ebase = (cid * NSUB + sid) * EDGES_PER_W

        @pl.loop(0, NBLK)
        def _(g):
            base = ebase + g * BLK
            pltpu.sync_copy(dst_hbm.at[pl.ds(base, BLK)], dst_v)
            pltpu.sync_copy(ones_v, cnt_sh.at[dst_v], add=True)

        plsc.subcore_barrier()

        out0 = cid * NPAD + row0
        for kk in range(NCH):
            pltpu.sync_copy(cnt_sh.at[pl.ds(row0 + kk * CH, CH), :],
                            cnt_hbm.at[pl.ds(out0 + kk * CH, CH), :])

    return k(dst)


def _edge_stage(e_feats, g, we1, be1, we2, be2, wf):
    def body(e_ref, g_ref, w1_ref, b1_ref, w2_ref, b2_ref, wf_ref, o_ref):
        h = jnp.maximum(
            jnp.dot(e_ref[...], w1_ref[...], preferred_element_type=jnp.float32)
            + b1_ref[...], 0.0)
        w = jnp.dot(h, w2_ref[...], preferred_element_type=jnp.float32) + b2_ref[...]
        o_ref[...] = jnp.dot(w * g_ref[...], wf_ref[...],
                             preferred_element_type=jnp.float32)

    be = 2000
    return pl.pallas_call(
        body,
        grid=(N_EDGES // be,),
        in_specs=[
            pl.BlockSpec((be, D_EDGE), lambda i: (i, 0)),
            pl.BlockSpec((be, W3), lambda i: (i, 0)),
            pl.BlockSpec((D_EDGE, UNITS), lambda i: (0, 0)),
            pl.BlockSpec((1, UNITS), lambda i: (0, 0)),
            pl.BlockSpec((UNITS, W3), lambda i: (0, 0)),
            pl.BlockSpec((1, W3), lambda i: (0, 0)),
            pl.BlockSpec((W3, UNITS), lambda i: (0, 0)),
        ],
        out_specs=pl.BlockSpec((be, UNITS), lambda i: (i, 0)),
        out_shape=jax.ShapeDtypeStruct((N_EDGES, UNITS), jnp.float32),
    )(e_feats, g, we1, be1, we2, be2, wf)


def _sc_scatter(z, dst):
    mesh = plsc.VectorSubcoreMesh(core_axis_name="c", subcore_axis_name="s")

    @functools.partial(
        pl.kernel,
        mesh=mesh,
        out_type=jax.ShapeDtypeStruct((NCORE * NPAD, 128), jnp.float32),
        scratch_types=[
            pltpu.VMEM((BLK,), jnp.int32),
            pltpu.VMEM((BLK, 128), jnp.float32),
            pltpu.VMEM((CH, 128), jnp.float32),
            pltpu.VMEM_SHARED((NPAD, 128), jnp.float32),
        ],
    )
    def k(z_hbm, dst_hbm, acc_hbm, dst_v, z_v, zz_v, acc_sh):
        cid = lax.axis_index("c")
        sid = lax.axis_index("s")

        @pl.loop(0, CH)
        def _(r):
            @pl.loop(0, 128, step=16)
            def _(c0):
                zz_v.at[r, pl.ds(c0, 16)][...] = jnp.zeros((16,), jnp.float32)

        row0 = sid * STRIPE
        for kk in range(NCH):
            pltpu.sync_copy(zz_v, acc_sh.at[pl.ds(row0 + kk * CH, CH), :])
        plsc.subcore_barrier()

        ebase = (cid * NSUB + sid) * EDGES_PER_W

        @pl.loop(0, NBLK)
        def _(g):
            base = ebase + g * BLK
            pltpu.sync_copy(dst_hbm.at[pl.ds(base, BLK)], dst_v)
            pltpu.sync_copy(z_hbm.at[pl.ds(base, BLK), :], z_v)
            pltpu.sync_copy(z_v, acc_sh.at[dst_v], add=True)

        plsc.subcore_barrier()

        out0 = cid * NPAD + row0
        for kk in range(NCH):
            pltpu.sync_copy(acc_sh.at[pl.ds(row0 + kk * CH, CH), :],
                            acc_hbm.at[pl.ds(out0 + kk * CH, CH), :])

    return k(z, dst)


def _fusion(acc0, acc1, cnt0, cnt1, bf2):
    def body(a0_ref, a1_ref, c0_ref, c1_ref, b_ref, o_ref):
        s = a0_ref[...] + a1_ref[...]
        counts = c0_ref[...][:, 0:1] + c1_ref[...][:, 0:1]
        denom = jnp.maximum(counts, 1.0)
        o_ref[...] = jnp.maximum(s / denom + b_ref[...], 0.0)

    bn = 1024
    return pl.pallas_call(
        body,
        grid=(NPAD // bn,),
        in_specs=[
            pl.BlockSpec((bn, UNITS), lambda i: (i, 0)),
            pl.BlockSpec((bn, UNITS), lambda i: (i, 0)),
            pl.BlockSpec((bn, 128), lambda i: (i, 0)),
            pl.BlockSpec((bn, 128), lambda i: (i, 0)),
            pl.BlockSpec((1, UNITS), lambda i: (0, 0)),
        ],
        out_specs=pl.BlockSpec((bn, UNITS), lambda i: (i, 0)),
        out_shape=jax.ShapeDtypeStruct((NPAD, UNITS), jnp.float32),
    )(acc0, acc1, cnt0, cnt1, bf2)


@jax.jit
def kernel(node_features, edge_indices, edge_features,
           K0, K1, K2, We1, be1, We2, be2, Wf, bf):
    src = edge_indices[0].astype(jnp.int32)
    dst = edge_indices[1].astype(jnp.int32)

    kcat = jnp.concatenate([K0, K1, K2], axis=1)

    y = _node_tables(node_features, kcat)
    g = _sc_gather(y, src)
    cnt = _sc_counts(dst)
    z = _edge_stage(edge_features, g, We1, be1.reshape(1, UNITS),
                    We2, be2.reshape(1, W3), Wf)
    acc = _sc_scatter(z, dst)

    out = _fusion(acc[:NPAD], acc[NPAD:], cnt[:NPAD], cnt[NPAD:],
                  bf.reshape(1, UNITS))
    return out[:N_NODES]


# speedup vs baseline: 4.7649x; 1.1266x over previous
import functools

import jax
import jax.numpy as jnp
from jax import lax
from jax.experimental import pallas as pl
from jax.experimental.pallas import tpu as pltpu
from jax.experimental.pallas import tpu_sc as plsc

N_NODES = 10000
N_EDGES = 320000
D_FEAT = 128
D_EDGE = 16
UNITS = 128
W3 = 384

NCORE = 2
NSUB = 16
NWORK = NCORE * NSUB
EDGES_PER_W = N_EDGES // NWORK
BLK = 80
NBLK = EDGES_PER_W // BLK
NPAD = 10240
STRIPE = NPAD // NSUB
CH = 32
NCH = STRIPE // CH


def _node_tables(x, kcat):
    def body(x_ref, k_ref, o_ref):
        o_ref[...] = jnp.dot(x_ref[...], k_ref[...],
                             preferred_element_type=jnp.float32)

    bn = 1000
    return pl.pallas_call(
        body,
        grid=(N_NODES // bn,),
        in_specs=[
            pl.BlockSpec((bn, D_FEAT), lambda i: (i, 0)),
            pl.BlockSpec((D_FEAT, W3), lambda i: (0, 0)),
        ],
        out_specs=pl.BlockSpec((bn, W3), lambda i: (i, 0)),
        out_shape=jax.ShapeDtypeStruct((N_NODES, W3), jnp.float32),
    )(x, kcat)


def _sc_gather(y, src):
    mesh = plsc.VectorSubcoreMesh(core_axis_name="c", subcore_axis_name="s")

    @functools.partial(
        pl.kernel,
        mesh=mesh,
        out_type=jax.ShapeDtypeStruct((N_EDGES, W3), jnp.float32),
        scratch_types=[
            pltpu.VMEM((3, BLK), jnp.int32),
            pltpu.VMEM((BLK, W3), jnp.float32),
            pltpu.VMEM((BLK, W3), jnp.float32),
            pltpu.VMEM((BLK, W3), jnp.float32),
            pltpu.SemaphoreType.DMA,
            pltpu.SemaphoreType.DMA,
        ],
    )
    def k(y_hbm, src_hbm, g_hbm, src_r, g_v0, g_v1, g_v2, sem_g, sem_w):
        cid = lax.axis_index("c")
        sid = lax.axis_index("s")
        ebase = (cid * NSUB + sid) * EDGES_PER_W
        gs = (g_v0, g_v1, g_v2)

        def fire_gather(b, r):
            pltpu.sync_copy(src_hbm.at[pl.ds(ebase + b * BLK, BLK)],
                            src_r.at[r])
            pltpu.async_copy(y_hbm.at[src_r.at[r]], gs[r], sem_g)

        def wait_gather(r):
            pltpu.make_async_copy(y_hbm.at[src_r.at[r]], gs[r], sem_g).wait()

        def fire_write(b, r):
            pltpu.async_copy(gs[r], g_hbm.at[pl.ds(ebase + b * BLK, BLK), :],
                             sem_w)

        def wait_write(b, r):
            pltpu.make_async_copy(
                gs[r], g_hbm.at[pl.ds(ebase + b * BLK, BLK), :], sem_w).wait()

        fire_gather(0, 0)
        fire_gather(1, 1)
        wait_gather(0); fire_write(0, 0); fire_gather(2, 2)
        wait_gather(1); fire_write(1, 1); wait_write(0, 0); fire_gather(3, 0)
        wait_gather(2); fire_write(2, 2); wait_write(1, 1); fire_gather(4, 1)

        @pl.loop(3, NBLK - 2, step=3)
        def _(g):
            for r in range(3):
                b = g + r
                wait_gather(r)
                fire_write(b, r)
                wait_write(b - 1, (r + 2) % 3)
                fire_gather(b + 2, (r + 2) % 3)

        wait_gather(0); fire_write(NBLK - 2, 0); wait_write(NBLK - 3, 2)
        wait_gather(1); fire_write(NBLK - 1, 1); wait_write(NBLK - 2, 0)
        wait_write(NBLK - 1, 1)

    return k(y, src)


def _sc_counts(dst):
    mesh = plsc.VectorSubcoreMesh(core_axis_name="c", subcore_axis_name="s")

    @functools.partial(
        pl.kernel,
        mesh=mesh,
        out_type=jax.ShapeDtypeStruct((NCORE * NPAD, 128), jnp.float32),
        scratch_types=[
            pltpu.VMEM((4, BLK), jnp.int32),
            pltpu.VMEM((BLK, 128), jnp.float32),
            pltpu.VMEM((CH, 128), jnp.float32),
            pltpu.VMEM_SHARED((NPAD, 128), jnp.float32),
            pltpu.SemaphoreType.DMA,
        ],
    )
    def k(dst_hbm, cnt_hbm, dst_r, ones_v, z_v, cnt_sh, sem):
        cid = lax.axis_index("c")
        sid = lax.axis_index("s")

        @pl.loop(0, CH)
        def _(r):
            @pl.loop(0, 128, step=16)
            def _(c0):
                z_v.at[r, pl.ds(c0, 16)][...] = jnp.zeros((16,), jnp.float32)

        @pl.loop(0, BLK)
        def _(r):
            @pl.loop(0, 128, step=16)
            def _(c0):
                ones_v.at[r, pl.ds(c0, 16)][...] = jnp.ones((16,), jnp.float32)

        row0 = sid * STRIPE
        for kk in range(NCH):
            pltpu.sync_copy(z_v, cnt_sh.at[pl.ds(row0 + kk * CH, CH), :])
        plsc.subcore_barrier()

        ebase = (cid * NSUB + sid) * EDGES_PER_W

        def fire(b, r):
            pltpu.sync_copy(dst_hbm.at[pl.ds(ebase + b * BLK, BLK)],
                            dst_r.at[r])
            pltpu.async_copy(ones_v, cnt_sh.at[dst_r.at[r]], sem, add=True)

        def wait_one(r):
            pltpu.make_async_copy(ones_v, cnt_sh.at[dst_r.at[r]], sem).wait()

        for b in range(4):
            fire(b, b)

        @pl.loop(4, NBLK - 1, step=4)
        def _(g):
            for r in range(4):
                wait_one(r)
                fire(g + r, r)

        wait_one(0)
        fire(NBLK - 1, 0)
        for r in (1, 2, 3, 0):
            wait_one(r)

        plsc.subcore_barrier()

        out0 = cid * NPAD + row0
        for kk in range(NCH):
            pltpu.sync_copy(cnt_sh.at[pl.ds(row0 + kk * CH, CH), :],
                            cnt_hbm.at[pl.ds(out0 + kk * CH, CH), :])

    return k(dst)


def _edge_stage(e_feats, g, we1, be1, we2, be2, wf):
    def body(e_ref, g_ref, w1_ref, b1_ref, w2_ref, b2_ref, wf_ref, o_ref):
        h = jnp.maximum(
            jnp.dot(e_ref[...], w1_ref[...], preferred_element_type=jnp.float32)
            + b1_ref[...], 0.0)
        w = jnp.dot(h, w2_ref[...], preferred_element_type=jnp.float32) + b2_ref[...]
        o_ref[...] = jnp.dot(w * g_ref[...], wf_ref[...],
                             preferred_element_type=jnp.float32)

    be = 2000
    return pl.pallas_call(
        body,
        grid=(N_EDGES // be,),
        in_specs=[
            pl.BlockSpec((be, D_EDGE), lambda i: (i, 0)),
            pl.BlockSpec((be, W3), lambda i: (i, 0)),
            pl.BlockSpec((D_EDGE, UNITS), lambda i: (0, 0)),
            pl.BlockSpec((1, UNITS), lambda i: (0, 0)),
            pl.BlockSpec((UNITS, W3), lambda i: (0, 0)),
            pl.BlockSpec((1, W3), lambda i: (0, 0)),
            pl.BlockSpec((W3, UNITS), lambda i: (0, 0)),
        ],
        out_specs=pl.BlockSpec((be, UNITS), lambda i: (i, 0)),
        out_shape=jax.ShapeDtypeStruct((N_EDGES, UNITS), jnp.float32),
    )(e_feats, g, we1, be1, we2, be2, wf)


def _sc_scatter(z, dst):
    mesh = plsc.VectorSubcoreMesh(core_axis_name="c", subcore_axis_name="s")

    @functools.partial(
        pl.kernel,
        mesh=mesh,
        out_type=jax.ShapeDtypeStruct((NCORE * NPAD, 128), jnp.float32),
        scratch_types=[
            pltpu.VMEM((4, BLK), jnp.int32),
            pltpu.VMEM((4, BLK, 128), jnp.float32),
            pltpu.VMEM((CH, 128), jnp.float32),
            pltpu.VMEM_SHARED((NPAD, 128), jnp.float32),
            pltpu.SemaphoreType.DMA,
            pltpu.SemaphoreType.DMA,
        ],
    )
    def k(z_hbm, dst_hbm, acc_hbm, dst_r, z_r, zz_v, acc_sh, sem_l, sem_s):
        cid = lax.axis_index("c")
        sid = lax.axis_index("s")

        @pl.loop(0, CH)
        def _(r):
            @pl.loop(0, 128, step=16)
            def _(c0):
                zz_v.at[r, pl.ds(c0, 16)][...] = jnp.zeros((16,), jnp.float32)

        row0 = sid * STRIPE
        for kk in range(NCH):
            pltpu.sync_copy(zz_v, acc_sh.at[pl.ds(row0 + kk * CH, CH), :])
        plsc.subcore_barrier()

        ebase = (cid * NSUB + sid) * EDGES_PER_W

        def fire_load(b, r):
            pltpu.sync_copy(dst_hbm.at[pl.ds(ebase + b * BLK, BLK)],
                            dst_r.at[r])
            pltpu.async_copy(z_hbm.at[pl.ds(ebase + b * BLK, BLK), :],
                             z_r.at[r], sem_l)

        def wait_load(b, r):
            pltpu.make_async_copy(
                z_hbm.at[pl.ds(ebase + b * BLK, BLK), :], z_r.at[r],
                sem_l).wait()

        def fire_scat(r):
            pltpu.async_copy(z_r.at[r], acc_sh.at[dst_r.at[r]], sem_s,
                             add=True)

        def wait_scat(r):
            pltpu.make_async_copy(z_r.at[r], acc_sh.at[dst_r.at[r]],
                                  sem_s).wait()

        for b in range(4):
            fire_load(b, b)

        @pl.loop(0, NBLK - 5, step=4)
        def _(g):
            for r in range(4):
                b = g + r
                wait_load(b, r)
                fire_scat(r)
                wait_scat(r)
                fire_load(b + 4, r)

        wait_load(NBLK - 5, 0)
        fire_scat(0)
        wait_scat(0)
        fire_load(NBLK - 1, 0)
        for r, b in ((1, NBLK - 4), (2, NBLK - 3), (3, NBLK - 2),
                     (0, NBLK - 1)):
            wait_load(b, r)
            fire_scat(r)
            wait_scat(r)

        plsc.subcore_barrier()

        out0 = cid * NPAD + row0
        for kk in range(NCH):
            pltpu.sync_copy(acc_sh.at[pl.ds(row0 + kk * CH, CH), :],
                            acc_hbm.at[pl.ds(out0 + kk * CH, CH), :])

    return k(z, dst)


def _fusion(acc0, acc1, cnt0, cnt1, bf2):
    def body(a0_ref, a1_ref, c0_ref, c1_ref, b_ref, o_ref):
        s = a0_ref[...] + a1_ref[...]
        counts = c0_ref[...][:, 0:1] + c1_ref[...][:, 0:1]
        denom = jnp.maximum(counts, 1.0)
        o_ref[...] = jnp.maximum(s / denom + b_ref[...], 0.0)

    bn = 1024
    return pl.pallas_call(
        body,
        grid=(NPAD // bn,),
        in_specs=[
            pl.BlockSpec((bn, UNITS), lambda i: (i, 0)),
            pl.BlockSpec((bn, UNITS), lambda i: (i, 0)),
            pl.BlockSpec((bn, 128), lambda i: (i, 0)),
            pl.BlockSpec((bn, 128), lambda i: (i, 0)),
            pl.BlockSpec((1, UNITS), lambda i: (0, 0)),
        ],
        out_specs=pl.BlockSpec((bn, UNITS), lambda i: (i, 0)),
        out_shape=jax.ShapeDtypeStruct((NPAD, UNITS), jnp.float32),
    )(acc0, acc1, cnt0, cnt1, bf2)


@jax.jit
def kernel(node_features, edge_indices, edge_features,
           K0, K1, K2, We1, be1, We2, be2, Wf, bf):
    src = edge_indices[0].astype(jnp.int32)
    dst = edge_indices[1].astype(jnp.int32)

    kcat = jnp.concatenate([K0, K1, K2], axis=1)

    y = _node_tables(node_features, kcat)
    g = _sc_gather(y, src)
    cnt = _sc_counts(dst)
    z = _edge_stage(edge_features, g, We1, be1.reshape(1, UNITS),
                    We2, be2.reshape(1, W3), Wf)
    acc = _sc_scatter(z, dst)

    out = _fusion(acc[:NPAD], acc[NPAD:], cnt[:NPAD], cnt[NPAD:],
                  bf.reshape(1, UNITS))
    return out[:N_NODES]


# speedup vs baseline: 6.3692x; 1.3367x over previous
import functools

import jax
import jax.numpy as jnp
from jax import lax
from jax.experimental import pallas as pl
from jax.experimental.pallas import tpu as pltpu
from jax.experimental.pallas import tpu_sc as plsc

N_NODES = 10000
N_EDGES = 320000
D_FEAT = 128
D_EDGE = 16
UNITS = 128
W3 = 384

NCORE = 2
NSUB = 16
NWORK = NCORE * NSUB
EDGES_PER_W = N_EDGES // NWORK
BLK = 80
NBLK = EDGES_PER_W // BLK
NPAD = 10240
STRIPE = NPAD // NSUB
CH = 32
NCH = STRIPE // CH


def _sc_gather_counts(x, src, dst):
    mesh = plsc.VectorSubcoreMesh(core_axis_name="c", subcore_axis_name="s")

    @functools.partial(
        pl.kernel,
        mesh=mesh,
        out_type=[
            jax.ShapeDtypeStruct((N_EDGES, D_FEAT), jnp.float32),
            jax.ShapeDtypeStruct((NCORE * NPAD, 128), jnp.float32),
        ],
        scratch_types=[
            pltpu.VMEM((3, BLK), jnp.int32),
            pltpu.VMEM((3, BLK), jnp.int32),
            pltpu.VMEM((BLK, D_FEAT), jnp.float32),
            pltpu.VMEM((BLK, D_FEAT), jnp.float32),
            pltpu.VMEM((BLK, D_FEAT), jnp.float32),
            pltpu.VMEM((BLK, 128), jnp.float32),
            pltpu.VMEM((CH, 128), jnp.float32),
            pltpu.VMEM_SHARED((NPAD, 128), jnp.float32),
            pltpu.SemaphoreType.DMA,
            pltpu.SemaphoreType.DMA,
            pltpu.SemaphoreType.DMA,
        ],
    )
    def k(x_hbm, src_hbm, dst_hbm, g_hbm, cnt_hbm,
          src_r, dst_r, g_v0, g_v1, g_v2, ones_v, z_v, cnt_sh,
          sem_g, sem_w, sem_c):
        cid = lax.axis_index("c")
        sid = lax.axis_index("s")
        ebase = (cid * NSUB + sid) * EDGES_PER_W
        gs = (g_v0, g_v1, g_v2)

        @pl.loop(0, CH)
        def _(r):
            @pl.loop(0, 128, step=16)
            def _(c0):
                z_v.at[r, pl.ds(c0, 16)][...] = jnp.zeros((16,), jnp.float32)

        @pl.loop(0, BLK)
        def _(r):
            @pl.loop(0, 128, step=16)
            def _(c0):
                ones_v.at[r, pl.ds(c0, 16)][...] = jnp.ones((16,), jnp.float32)

        row0 = sid * STRIPE
        for kk in range(NCH):
            pltpu.sync_copy(z_v, cnt_sh.at[pl.ds(row0 + kk * CH, CH), :])
        plsc.subcore_barrier()

        def fire_gather(b, r):
            pltpu.sync_copy(src_hbm.at[pl.ds(ebase + b * BLK, BLK)],
                            src_r.at[r])
            pltpu.async_copy(x_hbm.at[src_r.at[r]], gs[r], sem_g)

        def wait_gather(r):
            pltpu.make_async_copy(x_hbm.at[src_r.at[r]], gs[r], sem_g).wait()

        def fire_write(b, r):
            pltpu.async_copy(gs[r], g_hbm.at[pl.ds(ebase + b * BLK, BLK), :],
                             sem_w)

        def wait_write(b, r):
            pltpu.make_async_copy(
                gs[r], g_hbm.at[pl.ds(ebase + b * BLK, BLK), :], sem_w).wait()

        def fire_cnt(b, r):
            pltpu.sync_copy(dst_hbm.at[pl.ds(ebase + b * BLK, BLK)],
                            dst_r.at[r])
            pltpu.async_copy(ones_v, cnt_sh.at[dst_r.at[r]], sem_c, add=True)

        def wait_cnt(r):
            pltpu.make_async_copy(ones_v, cnt_sh.at[dst_r.at[r]],
                                  sem_c).wait()

        fire_gather(0, 0); fire_cnt(0, 0)
        fire_gather(1, 1); fire_cnt(1, 1)
        wait_gather(0); fire_write(0, 0); fire_gather(2, 2); fire_cnt(2, 2)
        wait_gather(1); fire_write(1, 1); wait_write(0, 0)
        fire_gather(3, 0); wait_cnt(0); fire_cnt(3, 0)
        wait_gather(2); fire_write(2, 2); wait_write(1, 1)
        fire_gather(4, 1); wait_cnt(1); fire_cnt(4, 1)

        @pl.loop(3, NBLK - 2, step=3)
        def _(g):
            for r in range(3):
                b = g + r
                r2 = (r + 2) % 3
                wait_gather(r)
                fire_write(b, r)
                wait_write(b - 1, r2)
                fire_gather(b + 2, r2)
                wait_cnt(r2)
                fire_cnt(b + 2, r2)

        wait_gather(0); fire_write(NBLK - 2, 0); wait_write(NBLK - 3, 2)
        wait_gather(1); fire_write(NBLK - 1, 1); wait_write(NBLK - 2, 0)
        wait_write(NBLK - 1, 1)
        wait_cnt(2); wait_cnt(0); wait_cnt(1)

        plsc.subcore_barrier()

        out0 = cid * NPAD + row0
        for kk in range(NCH):
            pltpu.sync_copy(cnt_sh.at[pl.ds(row0 + kk * CH, CH), :],
                            cnt_hbm.at[pl.ds(out0 + kk * CH, CH), :])

    return k(x, src, dst)


def _edge_stage(e_feats, g, we1, be1, we2, be2, kcat, wf):
    def body(e_ref, g_ref, w1_ref, b1_ref, w2_ref, b2_ref, kc_ref, wf_ref,
             o_ref):
        h = jnp.maximum(
            jnp.dot(e_ref[...], w1_ref[...], preferred_element_type=jnp.float32)
            + b1_ref[...], 0.0)
        w = jnp.dot(h, w2_ref[...], preferred_element_type=jnp.float32) + b2_ref[...]
        gk = jnp.dot(g_ref[...], kc_ref[...], preferred_element_type=jnp.float32)
        o_ref[...] = jnp.dot(w * gk, wf_ref[...],
                             preferred_element_type=jnp.float32)

    be = 2000
    return pl.pallas_call(
        body,
        grid=(N_EDGES // be,),
        in_specs=[
            pl.BlockSpec((be, D_EDGE), lambda i: (i, 0)),
            pl.BlockSpec((be, D_FEAT), lambda i: (i, 0)),
            pl.BlockSpec((D_EDGE, UNITS), lambda i: (0, 0)),
            pl.BlockSpec((1, UNITS), lambda i: (0, 0)),
            pl.BlockSpec((UNITS, W3), lambda i: (0, 0)),
            pl.BlockSpec((1, W3), lambda i: (0, 0)),
            pl.BlockSpec((D_FEAT, W3), lambda i: (0, 0)),
            pl.BlockSpec((W3, UNITS), lambda i: (0, 0)),
        ],
        out_specs=pl.BlockSpec((be, UNITS), lambda i: (i, 0)),
        out_shape=jax.ShapeDtypeStruct((N_EDGES, UNITS), jnp.float32),
    )(e_feats, g, we1, be1, we2, be2, kcat, wf)


def _sc_scatter(z, dst):
    mesh = plsc.VectorSubcoreMesh(core_axis_name="c", subcore_axis_name="s")

    @functools.partial(
        pl.kernel,
        mesh=mesh,
        out_type=jax.ShapeDtypeStruct((NCORE * NPAD, 128), jnp.float32),
        scratch_types=[
            pltpu.VMEM((4, BLK), jnp.int32),
            pltpu.VMEM((4, BLK, 128), jnp.float32),
            pltpu.VMEM((CH, 128), jnp.float32),
            pltpu.VMEM_SHARED((NPAD, 128), jnp.float32),
            pltpu.SemaphoreType.DMA,
            pltpu.SemaphoreType.DMA,
        ],
    )
    def k(z_hbm, dst_hbm, acc_hbm, dst_r, z_r, zz_v, acc_sh, sem_l, sem_s):
        cid = lax.axis_index("c")
        sid = lax.axis_index("s")

        @pl.loop(0, CH)
        def _(r):
            @pl.loop(0, 128, step=16)
            def _(c0):
                zz_v.at[r, pl.ds(c0, 16)][...] = jnp.zeros((16,), jnp.float32)

        row0 = sid * STRIPE
        for kk in range(NCH):
            pltpu.sync_copy(zz_v, acc_sh.at[pl.ds(row0 + kk * CH, CH), :])
        plsc.subcore_barrier()

        ebase = (cid * NSUB + sid) * EDGES_PER_W

        def fire_load(b, r):
            pltpu.sync_copy(dst_hbm.at[pl.ds(ebase + b * BLK, BLK)],
                            dst_r.at[r])
            pltpu.async_copy(z_hbm.at[pl.ds(ebase + b * BLK, BLK), :],
                             z_r.at[r], sem_l)

        def wait_load(b, r):
            pltpu.make_async_copy(
                z_hbm.at[pl.ds(ebase + b * BLK, BLK), :], z_r.at[r],
                sem_l).wait()

        def fire_scat(r):
            pltpu.async_copy(z_r.at[r], acc_sh.at[dst_r.at[r]], sem_s,
                             add=True)

        def wait_scat(r):
            pltpu.make_async_copy(z_r.at[r], acc_sh.at[dst_r.at[r]],
                                  sem_s).wait()

        for b in range(4):
            fire_load(b, b)

        @pl.loop(0, NBLK - 5, step=4)
        def _(g):
            for r in range(4):
                b = g + r
                wait_load(b, r)
                fire_scat(r)
                wait_scat(r)
                fire_load(b + 4, r)

        wait_load(NBLK - 5, 0)
        fire_scat(0)
        wait_scat(0)
        fire_load(NBLK - 1, 0)
        for r, b in ((1, NBLK - 4), (2, NBLK - 3), (3, NBLK - 2),
                     (0, NBLK - 1)):
            wait_load(b, r)
            fire_scat(r)
            wait_scat(r)

        plsc.subcore_barrier()

        out0 = cid * NPAD + row0
        for kk in range(NCH):
            pltpu.sync_copy(acc_sh.at[pl.ds(row0 + kk * CH, CH), :],
                            acc_hbm.at[pl.ds(out0 + kk * CH, CH), :])

    return k(z, dst)


def _fusion(acc0, acc1, cnt0, cnt1, bf2):
    def body(a0_ref, a1_ref, c0_ref, c1_ref, b_ref, o_ref):
        s = a0_ref[...] + a1_ref[...]
        counts = c0_ref[...][:, 0:1] + c1_ref[...][:, 0:1]
        denom = jnp.maximum(counts, 1.0)
        o_ref[...] = jnp.maximum(s / denom + b_ref[...], 0.0)

    bn = 1024
    return pl.pallas_call(
        body,
        grid=(NPAD // bn,),
        in_specs=[
            pl.BlockSpec((bn, UNITS), lambda i: (i, 0)),
            pl.BlockSpec((bn, UNITS), lambda i: (i, 0)),
            pl.BlockSpec((bn, 128), lambda i: (i, 0)),
            pl.BlockSpec((bn, 128), lambda i: (i, 0)),
            pl.BlockSpec((1, UNITS), lambda i: (0, 0)),
        ],
        out_specs=pl.BlockSpec((bn, UNITS), lambda i: (i, 0)),
        out_shape=jax.ShapeDtypeStruct((NPAD, UNITS), jnp.float32),
    )(acc0, acc1, cnt0, cnt1, bf2)


@jax.jit
def kernel(node_features, edge_indices, edge_features,
           K0, K1, K2, We1, be1, We2, be2, Wf, bf):
    src = edge_indices[0].astype(jnp.int32)
    dst = edge_indices[1].astype(jnp.int32)

    kcat = jnp.concatenate([K0, K1, K2], axis=1)

    g, cnt = _sc_gather_counts(node_features, src, dst)
    z = _edge_stage(edge_features, g, We1, be1.reshape(1, UNITS),
                    We2, be2.reshape(1, W3), kcat, Wf)
    acc = _sc_scatter(z, dst)

    out = _fusion(acc[:NPAD], acc[NPAD:], cnt[:NPAD], cnt[NPAD:],
                  bf.reshape(1, UNITS))
    return out[:N_NODES]
